# Initial kernel scaffold; baseline (speedup 1.0000x reference)
import jax
import jax.numpy as jnp
from jax.experimental import pallas as pl


def kernel(x, W):
    raise NotImplementedError("write your pallas kernel here")


# speedup vs baseline: 13.1206x; 13.1206x over previous
import math

import jax
import jax.numpy as jnp
from jax import lax
from jax.experimental import pallas as pl
from jax.experimental.pallas import tpu as pltpu
from jax.experimental.pallas import tpu_sc as plsc

_NCB = 4
_V = 100000
_D = 128
_B = 1024
_S = 200
_N = _B * _S
_NC = 2
_NS = 16
_NW = _NC * _NS
_RW = _N // _NW
_C = 64
_CHUNKS = _RW // _C
_LANES = 16
_VPR = _D // _LANES
_SCALE = math.sqrt(_D)


def _sc_body(idx_hbm, w_hbm, out_hbm,
             idx_v, gb0, gb1, ob0, ob1,
             gsem0, gsem1, osem0, osem1, isem):
    wid = lax.axis_index("s") * _NC + lax.axis_index("c")
    base = wid * _RW

    gbufs = (gb0, gb1)
    obufs = (ob0, ob1)
    gsems = (gsem0, gsem1)
    osems = (osem0, osem1)

    pltpu.async_copy(idx_hbm.at[wid], idx_v, isem).wait()

    def issue_gathers(g, b):
        for h in range(2):
            pltpu.async_copy(
                w_hbm.at[idx_v.at[g, h]],
                gbufs[b].at[pl.ds(h * 2 * _C, 2 * _C)],
                gsems[b])

    def wait_gathers(b):
        for h in range(2):
            pltpu.make_async_copy(
                w_hbm.at[pl.ds(0, 2 * _C)],
                gbufs[b].at[pl.ds(h * 2 * _C, 2 * _C)],
                gsems[b]).wait()

    def issue_store(g, b):
        pltpu.async_copy(
            obufs[b], out_hbm.at[pl.ds(base + g * _C, _C)], osems[b])

    def wait_store(b):
        pltpu.make_async_copy(
            obufs[b], out_hbm.at[pl.ds(0, _C)], osems[b]).wait()

    def compute(b):
        gb, ob = gbufs[b], obufs[b]

        @pl.loop(0, _C)
        def _(r):
            for c in range(_VPR):
                s = pl.ds(c * _LANES, _LANES)
                v = ((gb[r, s] + gb[_C + r, s])
                     + (gb[2 * _C + r, s] + gb[3 * _C + r, s]))
                ob[r, s] = v * _SCALE

    issue_gathers(0, 0)
    issue_gathers(1, 1)

    for b in range(2):
        wait_gathers(b)
        compute(b)
        issue_store(b, b)
        issue_gathers(b + 2, b)

    @pl.loop(2, _CHUNKS - 2, step=2)
    def _(g0):
        for b in range(2):
            g = g0 + b
            wait_gathers(b)
            wait_store(b)
            compute(b)
            issue_store(g, b)
            issue_gathers(g + 2, b)

    for b in range(2):
        g = _CHUNKS - 2 + b
        wait_gathers(b)
        wait_store(b)
        compute(b)
        issue_store(g, b)

    wait_store(0)
    wait_store(1)


def kernel(x, W):
    xt = jnp.transpose(x, (1, 0, 2)).reshape(_NCB, _N)
    idx_all = xt + (jnp.arange(_NCB, dtype=jnp.int32) * _V)[:, None]
    idx_r = (idx_all.reshape(_NCB, _NW, _CHUNKS, _C)
             .transpose(1, 2, 0, 3)
             .reshape(_NW, _CHUNKS, 2, 2 * _C))
    w_flat = W.reshape(_NCB * _V, _D)

    mesh = plsc.VectorSubcoreMesh(core_axis_name="c", subcore_axis_name="s")
    out = pl.kernel(
        _sc_body,
        out_type=jax.ShapeDtypeStruct((_N, _D), jnp.float32),
        mesh=mesh,
        scratch_types=[
            pltpu.VMEM((_CHUNKS, 2, 2 * _C), jnp.int32),
            pltpu.VMEM((_NCB * _C, _D), jnp.float32),
            pltpu.VMEM((_NCB * _C, _D), jnp.float32),
            pltpu.VMEM((_C, _D), jnp.float32),
            pltpu.VMEM((_C, _D), jnp.float32),
            pltpu.SemaphoreType.DMA,
            pltpu.SemaphoreType.DMA,
            pltpu.SemaphoreType.DMA,
            pltpu.SemaphoreType.DMA,
            pltpu.SemaphoreType.DMA,
        ],
    )(idx_r, w_flat)
    return out.reshape(_B, _S, _D)
